# baseline (speedup 1.0000x reference)
import dataclasses
import functools

import jax
import jax.numpy as jnp
from jax import lax
from jax.experimental import pallas as pl
from jax.experimental.pallas import tpu as pltpu
from jax.experimental.pallas import tpu_sc as plsc

_NC = 2
_NS = 16
_NW = _NC * _NS
_LANES = 16
_CHUNK = 128


def _scale_rows(blk, bufi, buf):
    @pl.loop(0, _CHUNK, step=_LANES)
    def _(e0):
        vv = plsc.bitcast(blk[2, pl.ds(e0, _LANES)], jnp.float32)
        for t in range(_LANES):
            v = vv[t]
            e = e0 + t
            for g in range(4):
                w = bufi[e, pl.ds(g * _LANES, _LANES)]
                ab = plsc.bitcast(w, jnp.bfloat16)
                a, b = plsc.unpack(ab, format=plsc.PackFormat.INTERLEAVED)
                buf[e, pl.ds(g * 32, _LANES)] = a * v
                buf[e, pl.ds(g * 32 + _LANES, _LANES)] = b * v


def _spmm_partials(X, idx_blocks, n_chunks):
    N, _ = X.shape
    D = 128
    per_tile = n_chunks // _NW
    n_main = per_tile * _NW
    n_left = n_chunks - n_main
    assert per_tile % 2 == 0 and n_left < _NW
    n_pairs = per_tile // 2

    rows_per_sub = (N // _NS) // 8 * 8
    tail_base = _NS * rows_per_sub
    tail_rows = N - tail_base

    mesh = plsc.VectorSubcoreMesh(core_axis_name="c", subcore_axis_name="s")
    cp = pltpu.CompilerParams()
    if "needs_layout_passes" in pltpu.CompilerParams.__dataclass_fields__:
        cp = dataclasses.replace(cp, needs_layout_passes=False)
    if "use_tc_tiling_on_sc" in pltpu.CompilerParams.__dataclass_fields__:
        cp = dataclasses.replace(cp, use_tc_tiling_on_sc=False)

    @functools.partial(
        pl.kernel,
        out_type=jax.ShapeDtypeStruct((_NC, N, D), jnp.float32),
        mesh=mesh,
        compiler_params=cp,
        scratch_types=[
            pltpu.VMEM((3, _CHUNK), jnp.int32),
            pltpu.VMEM((3, _CHUNK), jnp.int32),
            pltpu.VMEM((_CHUNK, 64), jnp.int32),
            pltpu.VMEM((_CHUNK, 64), jnp.int32),
            pltpu.VMEM((_CHUNK, 128), jnp.float32),
            pltpu.VMEM((_CHUNK, 128), jnp.float32),
            pltpu.VMEM_SHARED((N, 128), jnp.float32),
            pltpu.SemaphoreType.DMA,
            pltpu.SemaphoreType.DMA,
            pltpu.SemaphoreType.DMA,
            pltpu.SemaphoreType.DMA,
            pltpu.SemaphoreType.DMA,
            pltpu.SemaphoreType.DMA,
        ],
    )
    def sc_kernel(x_hbm, idx_hbm, out_hbm,
                  blk0, blk1, bufi0, bufi1, buf0, buf1, acc_sh,
                  sem_i0, sem_i1, sem_g0, sem_g1, sem_s0, sem_s1):
        cc = lax.axis_index("c")
        ss = lax.axis_index("s")
        wid = ss * _NC + cc
        base = ss * rows_per_sub

        zeros16 = jnp.zeros((_LANES,), jnp.float32)

        @pl.loop(0, _CHUNK)
        def _(r):
            for j in range(0, 128, _LANES):
                buf0[r, pl.ds(j, _LANES)] = zeros16

        off = 0
        while off < rows_per_sub:
            sz = min(_CHUNK, rows_per_sub - off)
            pltpu.sync_copy(buf0.at[pl.ds(0, sz)],
                            acc_sh.at[pl.ds(base + off, sz)])
            off += sz
        if tail_rows:
            @pl.when(ss == _NS - 1)
            def _():
                pltpu.sync_copy(buf0.at[pl.ds(0, tail_rows)],
                                acc_sh.at[pl.ds(tail_base, tail_rows)])
        plsc.subcore_barrier()

        def idx_start(k, blk, sem):
            return pltpu.async_copy(idx_hbm.at[wid + k * _NW], blk, sem)

        def idx_wait(blk, sem):
            pltpu.make_async_copy(idx_hbm.at[0], blk, sem).wait()

        def gather_start(blk, bufi, sem):
            return pltpu.async_copy(x_hbm.at[blk.at[0]], bufi, sem)

        def gather_wait(blk, bufi, sem):
            pltpu.make_async_copy(x_hbm.at[blk.at[0]], bufi, sem).wait()

        def scatter_start(blk, buf, sem):
            return pltpu.async_copy(buf, acc_sh.at[blk.at[1]], sem, add=True)

        idx_start(0, blk0, sem_i0).wait()
        gather_start(blk0, bufi0, sem_g0)
        idx_start(1, blk1, sem_i1)

        @pl.loop(0, n_pairs)
        def _(it):
            not_last = it < n_pairs - 1
            idx_wait(blk1, sem_i1)
            h_g1 = gather_start(blk1, bufi1, sem_g1)
            gather_wait(blk0, bufi0, sem_g0)
            _scale_rows(blk0, bufi0, buf0)
            h_s0 = scatter_start(blk0, buf0, sem_s0)
            h_g1.wait()
            _scale_rows(blk1, bufi1, buf1)
            h_s0.wait()

            @pl.when(not_last)
            def _():
                idx_start(2 * it + 2, blk0, sem_i0)

            h_s1 = scatter_start(blk1, buf1, sem_s1)

            @pl.when(not_last)
            def _():
                idx_wait(blk0, sem_i0)
                gather_start(blk0, bufi0, sem_g0)

            h_s1.wait()

            @pl.when(not_last)
            def _():
                idx_start(2 * it + 3, blk1, sem_i1)

        if n_left:
            @pl.when(wid < n_left)
            def _():
                pltpu.async_copy(idx_hbm.at[n_main + wid], blk0,
                                 sem_i0).wait()
                pltpu.async_copy(x_hbm.at[blk0.at[0]], bufi0, sem_g0).wait()
                _scale_rows(blk0, bufi0, buf0)
                pltpu.async_copy(buf0, acc_sh.at[blk0.at[1]], sem_s0,
                                 add=True).wait()

        plsc.subcore_barrier()

        pltpu.sync_copy(acc_sh.at[pl.ds(base, rows_per_sub)],
                        out_hbm.at[cc, pl.ds(base, rows_per_sub)])
        if tail_rows:
            @pl.when(ss == _NS - 1)
            def _():
                pltpu.sync_copy(acc_sh.at[pl.ds(tail_base, tail_rows)],
                                out_hbm.at[cc, pl.ds(tail_base, tail_rows)])

    return sc_kernel(X, idx_blocks)


def _finish(p0, p1, wt):
    N, D = p0.shape
    blk = 1000
    assert N % blk == 0

    def body(p0_ref, p1_ref, wt_ref, o_ref):
        acc = p0_ref[...] + p1_ref[...]
        o_ref[...] = jnp.dot(acc, wt_ref[...],
                             preferred_element_type=jnp.float32)

    return pl.pallas_call(
        body,
        grid=(N // blk,),
        in_specs=[
            pl.BlockSpec((blk, D), lambda i: (i, 0)),
            pl.BlockSpec((blk, D), lambda i: (i, 0)),
            pl.BlockSpec((D, D), lambda i: (0, 0)),
        ],
        out_specs=pl.BlockSpec((blk, D), lambda i: (i, 0)),
        out_shape=jax.ShapeDtypeStruct((N, D), jnp.float32),
    )(p0, p1, wt)


def kernel(X, W, rows, cols, vals):
    E = rows.shape[0]
    N = X.shape[0]
    assert E % _CHUNK == 0
    n_chunks = E // _CHUNK
    xb = X.reshape(N, 4, 2, 16).swapaxes(2, 3).astype(jnp.bfloat16)
    xb = jax.lax.bitcast_convert_type(xb, jnp.int32).reshape(N, 64)
    idx_blocks = jnp.stack(
        [
            cols.astype(jnp.int32).reshape(n_chunks, _CHUNK),
            rows.astype(jnp.int32).reshape(n_chunks, _CHUNK),
            jax.lax.bitcast_convert_type(vals, jnp.int32).reshape(
                n_chunks, _CHUNK),
        ],
        axis=1,
    )
    parts = _spmm_partials(xb, idx_blocks, n_chunks)
    return _finish(parts[0], parts[1], W.T)


# speedup vs baseline: 1.8342x; 1.8342x over previous
import dataclasses
import functools

import jax
import jax.numpy as jnp
from jax import lax
from jax.experimental import pallas as pl
from jax.experimental.pallas import tpu as pltpu
from jax.experimental.pallas import tpu_sc as plsc

_NC = 2
_NS = 16
_NW = _NC * _NS
_LANES = 16
_CHUNK = 128


def _scale_rows(blk, buf):
    @pl.loop(0, _CHUNK, step=_LANES)
    def _(e0):
        vv = plsc.bitcast(blk[2, pl.ds(e0, _LANES)], jnp.float32)
        for t in range(_LANES):
            v = vv[t]
            e = e0 + t
            for j in range(0, 128, _LANES):
                buf[e, pl.ds(j, _LANES)] = buf[e, pl.ds(j, _LANES)] * v


def _spmm_partials(X, idx_blocks, n_chunks):
    N, D = X.shape
    assert D == 128
    per_tile = n_chunks // _NW
    n_main = per_tile * _NW
    n_left = n_chunks - n_main
    assert per_tile % 3 == 0 and n_left < _NW

    rows_per_sub = (N // _NS) // 8 * 8
    tail_base = _NS * rows_per_sub
    tail_rows = N - tail_base

    mesh = plsc.VectorSubcoreMesh(core_axis_name="c", subcore_axis_name="s")
    cp = pltpu.CompilerParams()
    if "needs_layout_passes" in pltpu.CompilerParams.__dataclass_fields__:
        cp = dataclasses.replace(cp, needs_layout_passes=False)

    @functools.partial(
        pl.kernel,
        out_type=jax.ShapeDtypeStruct((_NC, N, D), jnp.float32),
        mesh=mesh,
        compiler_params=cp,
        scratch_types=[
            pltpu.VMEM((3, _CHUNK), jnp.int32),
            pltpu.VMEM((3, _CHUNK), jnp.int32),
            pltpu.VMEM((3, _CHUNK), jnp.int32),
            pltpu.VMEM((_CHUNK, 128), jnp.float32),
            pltpu.VMEM((_CHUNK, 128), jnp.float32),
            pltpu.VMEM((_CHUNK, 128), jnp.float32),
            pltpu.VMEM_SHARED((N, 128), jnp.float32),
            pltpu.SemaphoreType.DMA,
            pltpu.SemaphoreType.DMA,
            pltpu.SemaphoreType.DMA,
            pltpu.SemaphoreType.DMA,
            pltpu.SemaphoreType.DMA,
            pltpu.SemaphoreType.DMA,
            pltpu.SemaphoreType.DMA,
            pltpu.SemaphoreType.DMA,
            pltpu.SemaphoreType.DMA,
        ],
    )
    def sc_kernel(x_hbm, idx_hbm, out_hbm,
                  blkA, blkB, blkC, bufA, bufB, bufC, acc_sh,
                  sem_iA, sem_iB, sem_iC, sem_gA, sem_gB, sem_gC,
                  sem_sA, sem_sB, sem_sC):
        cc = lax.axis_index("c")
        ss = lax.axis_index("s")
        wid = ss * _NC + cc
        base = ss * rows_per_sub

        zeros16 = jnp.zeros((_LANES,), jnp.float32)

        @pl.loop(0, _CHUNK)
        def _(r):
            for j in range(0, 128, _LANES):
                bufA[r, pl.ds(j, _LANES)] = zeros16

        off = 0
        while off < rows_per_sub:
            sz = min(_CHUNK, rows_per_sub - off)
            pltpu.sync_copy(bufA.at[pl.ds(0, sz)],
                            acc_sh.at[pl.ds(base + off, sz)])
            off += sz
        if tail_rows:
            @pl.when(ss == _NS - 1)
            def _():
                pltpu.sync_copy(bufA.at[pl.ds(0, tail_rows)],
                                acc_sh.at[pl.ds(tail_base, tail_rows)])
        plsc.subcore_barrier()

        def idx_start(k, blk, sem):
            return pltpu.async_copy(idx_hbm.at[wid + k * _NW], blk, sem)

        def idx_wait(blk, sem):
            pltpu.make_async_copy(idx_hbm.at[0], blk, sem).wait()

        def gather_start(blk, buf, sem):
            return pltpu.async_copy(x_hbm.at[blk.at[0]], buf, sem)

        def gather_wait(blk, buf, sem):
            pltpu.make_async_copy(x_hbm.at[blk.at[0]], buf, sem).wait()

        def scatter_start(blk, buf, sem):
            return pltpu.async_copy(buf, acc_sh.at[blk.at[1]], sem, add=True)

        assert per_tile % 3 == 0
        n_trips = per_tile // 3

        idx_start(0, blkA, sem_iA).wait()
        gather_start(blkA, bufA, sem_gA)
        idx_start(1, blkB, sem_iB)
        idx_start(2, blkC, sem_iC)

        @pl.loop(0, n_trips)
        def _(it):
            not_last = it < n_trips - 1
            a = 3 * it
            idx_wait(blkC, sem_iC)
            gather_start(blkC, bufC, sem_gC)
            idx_wait(blkB, sem_iB)
            gather_start(blkB, bufB, sem_gB)
            gather_wait(blkA, bufA, sem_gA)
            _scale_rows(blkA, bufA)
            h_sA = scatter_start(blkA, bufA, sem_sA)
            h_sA.wait()

            @pl.when(not_last)
            def _():
                idx_start(a + 3, blkA, sem_iA)

            gather_wait(blkB, bufB, sem_gB)
            _scale_rows(blkB, bufB)
            h_sB = scatter_start(blkB, bufB, sem_sB)

            @pl.when(not_last)
            def _():
                idx_wait(blkA, sem_iA)
                gather_start(blkA, bufA, sem_gA)

            h_sB.wait()

            @pl.when(not_last)
            def _():
                idx_start(a + 4, blkB, sem_iB)

            gather_wait(blkC, bufC, sem_gC)
            _scale_rows(blkC, bufC)
            h_sC = scatter_start(blkC, bufC, sem_sC)
            h_sC.wait()

            @pl.when(not_last)
            def _():
                idx_start(a + 5, blkC, sem_iC)

        if n_left:
            @pl.when(wid < n_left)
            def _():
                pltpu.async_copy(idx_hbm.at[n_main + wid], blkA,
                                 sem_iA).wait()
                pltpu.async_copy(x_hbm.at[blkA.at[0]], bufA, sem_gA).wait()
                _scale_rows(blkA, bufA)
                pltpu.async_copy(bufA, acc_sh.at[blkA.at[1]], sem_sA,
                                 add=True).wait()

        plsc.subcore_barrier()

        pltpu.sync_copy(acc_sh.at[pl.ds(base, rows_per_sub)],
                        out_hbm.at[cc, pl.ds(base, rows_per_sub)])
        if tail_rows:
            @pl.when(ss == _NS - 1)
            def _():
                pltpu.sync_copy(acc_sh.at[pl.ds(tail_base, tail_rows)],
                                out_hbm.at[cc, pl.ds(tail_base, tail_rows)])

    return sc_kernel(X, idx_blocks)


def _finish(p0, p1, wt):
    N, D = p0.shape
    blk = 1000
    assert N % blk == 0

    def body(p0_ref, p1_ref, wt_ref, o_ref):
        acc = p0_ref[...] + p1_ref[...]
        o_ref[...] = jnp.dot(acc, wt_ref[...],
                             preferred_element_type=jnp.float32)

    return pl.pallas_call(
        body,
        grid=(N // blk,),
        in_specs=[
            pl.BlockSpec((blk, D), lambda i: (i, 0)),
            pl.BlockSpec((blk, D), lambda i: (i, 0)),
            pl.BlockSpec((D, D), lambda i: (0, 0)),
        ],
        out_specs=pl.BlockSpec((blk, D), lambda i: (i, 0)),
        out_shape=jax.ShapeDtypeStruct((N, D), jnp.float32),
    )(p0, p1, wt)


def kernel(X, W, rows, cols, vals):
    E = rows.shape[0]
    assert E % _CHUNK == 0
    n_chunks = E // _CHUNK
    idx_blocks = jnp.stack(
        [
            cols.astype(jnp.int32).reshape(n_chunks, _CHUNK),
            rows.astype(jnp.int32).reshape(n_chunks, _CHUNK),
            jax.lax.bitcast_convert_type(vals, jnp.int32).reshape(
                n_chunks, _CHUNK),
        ],
        axis=1,
    )
    parts = _spmm_partials(X, idx_blocks, n_chunks)
    return _finish(parts[0], parts[1], W.T)
